# baseline (speedup 1.0000x reference)
import functools

import jax
import jax.numpy as jnp
from jax import lax
from jax.experimental import pallas as pl
from jax.experimental.pallas import tpu as pltpu
from jax.experimental.pallas import tpu_sc as plsc

D_OUT = 64
SCALE = 8.0
BLK = 128
VSTAGE_BLK = 8192


@functools.cache
def _build_stage(V: int):
    VP = -(-V // VSTAGE_BLK) * VSTAGE_BLK

    def body(in_ref, out_ref):
        eye = jnp.eye(D_OUT, dtype=jnp.float32)
        blk = in_ref[...]
        y = lax.dot_general(
            blk, eye * SCALE, (((0,), (0,)), ((), ())),
            preferred_element_type=jnp.float32,
        )
        z = jnp.zeros((VSTAGE_BLK, D_OUT), jnp.float32)
        out_ref[...] = jnp.concatenate([y, z], axis=1)

    return pl.pallas_call(
        body,
        grid=(VP // VSTAGE_BLK,),
        in_specs=[pl.BlockSpec((D_OUT, VSTAGE_BLK), lambda i: (0, i))],
        out_specs=pl.BlockSpec((VSTAGE_BLK, 128), lambda i: (i, 0)),
        out_shape=jax.ShapeDtypeStruct((VP, 128), jnp.float32),
    )


@functools.cache
def _build_gather(B: int, VP: int):
    info = plsc.get_sparse_core_info()
    NC, NS = info.num_cores, info.num_subcores
    NW = NC * NS
    b_per_w = B // NW
    n_blocks = b_per_w // BLK
    mesh = plsc.VectorSubcoreMesh(core_axis_name="c", subcore_axis_name="s")

    def body(x_hbm, tab_hbm, out_hbm, idx_v, r0, r1, r2, r3,
             g0, g1, g2, g3, o0, o1, o2, o3):
        wid = lax.axis_index("s") * NC + lax.axis_index("c")
        base = wid * b_per_w
        pltpu.sync_copy(x_hbm.at[pl.ds(base, b_per_w)], idx_v)
        rows = [r0, r1, r2, r3]
        gsems = [g0, g1, g2, g3]
        osems = [o0, o1, o2, o3]

        def fire_gather(c, s):
            pltpu.async_copy(
                tab_hbm.at[idx_v.at[pl.ds(c * BLK, BLK)]], rows[s], gsems[s]
            )

        def drain_gather(c, s):
            pltpu.make_async_copy(
                tab_hbm.at[idx_v.at[pl.ds(c * BLK, BLK)]], rows[s], gsems[s]
            ).wait()

        def out_copy(c, s, fire):
            src = rows[s].at[pl.ds(0, BLK), pl.ds(0, D_OUT)]
            dst = out_hbm.at[pl.ds(base + c * BLK, BLK)]
            if fire:
                pltpu.async_copy(src, dst, osems[s])
            else:
                pltpu.make_async_copy(src, dst, osems[s]).wait()

        fire_gather(0, 0)
        fire_gather(1, 1)

        def group(g, carry):
            c0 = 4 * g
            for k in range(4):
                c = c0 + k

                @pl.when(c >= 2)
                def _():
                    out_copy(c - 2, (k + 2) % 4, False)

                @pl.when(c + 2 < n_blocks)
                def _():
                    fire_gather(c + 2, (k + 2) % 4)

                drain_gather(c, k)
                out_copy(c, k, True)
            return carry

        lax.fori_loop(0, n_blocks // 4, group, 0)
        out_copy(n_blocks - 2, (n_blocks - 2) % 4, False)
        out_copy(n_blocks - 1, (n_blocks - 1) % 4, False)

    return pl.kernel(
        body,
        out_type=jax.ShapeDtypeStruct((B, D_OUT), jnp.float32),
        mesh=mesh,
        compiler_params=pltpu.CompilerParams(
            use_tc_tiling_on_sc=False,
            needs_layout_passes=False,
            disable_bounds_checks=True,
        ),
        scratch_types=[
            pltpu.VMEM((b_per_w,), jnp.int32),
            pltpu.VMEM((BLK, 128), jnp.float32),
            pltpu.VMEM((BLK, 128), jnp.float32),
            pltpu.VMEM((BLK, 128), jnp.float32),
            pltpu.VMEM((BLK, 128), jnp.float32),
            pltpu.SemaphoreType.DMA,
            pltpu.SemaphoreType.DMA,
            pltpu.SemaphoreType.DMA,
            pltpu.SemaphoreType.DMA,
            pltpu.SemaphoreType.DMA,
            pltpu.SemaphoreType.DMA,
            pltpu.SemaphoreType.DMA,
            pltpu.SemaphoreType.DMA,
        ],
    )


@functools.cache
def _build_trans(S0: int, S1: int):
    NB = 32

    def body(in_ref, out_ref):
        eye = jnp.eye(D_OUT, dtype=jnp.float32)
        for b in range(NB):
            blk = in_ref[pl.ds(b * D_OUT, D_OUT), :]
            halves = [
                lax.dot_general(
                    blk[:, h * 64:(h + 1) * 64], eye,
                    (((0,), (0,)), ((), ())),
                    preferred_element_type=jnp.float32,
                )
                for h in range(2)
            ]
            y = jnp.concatenate(halves, axis=1)
            out_ref[0, :, b, :, :] = y.reshape(8, 8, BLK)

    return pl.pallas_call(
        body,
        grid=(S1, S0 // (BLK * NB)),
        in_specs=[
            pl.BlockSpec(
                (D_OUT * NB, 128),
                lambda s1, g: (s1 * (S0 // (BLK * NB)) + g, 0),
            )
        ],
        out_specs=pl.BlockSpec(
            (1, 8, NB, 8, BLK), lambda s1, g: (s1, 0, g, 0, 0)
        ),
        out_shape=jax.ShapeDtypeStruct(
            (S1, D_OUT // 8, S0 // BLK, 8, BLK), jnp.float32
        ),
    )


def kernel(x, table):
    s0, s1 = x.shape
    v, d = table.shape
    xt = x.T.reshape(s0 * s1).astype(jnp.int32)
    xt = xt.reshape(-1, 2, 64).transpose(0, 2, 1).reshape(s0 * s1)
    staged = _build_stage(v)(table.T)
    flat = _build_gather(s0 * s1, staged.shape[0])(xt, staged)
    out5d = _build_trans(s0, s1)(flat.reshape(s0 * s1 // 2, 128))
    return out5d.transpose(2, 4, 0, 1, 3).reshape(s0, s1, d)


# speedup vs baseline: 1.1778x; 1.1778x over previous
import functools

import jax
import jax.numpy as jnp
from jax import lax
from jax.experimental import pallas as pl
from jax.experimental.pallas import tpu as pltpu
from jax.experimental.pallas import tpu_sc as plsc

D_OUT = 64
SCALE = 8.0
BLK = 128
VSTAGE_BLK = 2048


@functools.cache
def _build_stage(V: int):
    VP = -(-V // VSTAGE_BLK) * VSTAGE_BLK

    def body(in_ref, out_ref):
        eye = jnp.eye(D_OUT, dtype=jnp.float32)
        blk = in_ref[...]
        y = lax.dot_general(
            blk, eye * SCALE, (((0,), (0,)), ((), ())),
            preferred_element_type=jnp.float32,
        )
        out_ref[:, 0:D_OUT] = y

    return pl.pallas_call(
        body,
        grid=(VP // VSTAGE_BLK,),
        in_specs=[pl.BlockSpec((D_OUT, VSTAGE_BLK), lambda i: (0, i))],
        out_specs=pl.BlockSpec((VSTAGE_BLK, 128), lambda i: (i, 0)),
        out_shape=jax.ShapeDtypeStruct((VP, 128), jnp.float32),
    )


@functools.cache
def _build_gather(B: int, VP: int):
    info = plsc.get_sparse_core_info()
    NC, NS = info.num_cores, info.num_subcores
    NW = NC * NS
    b_per_w = B // NW
    n_blocks = b_per_w // BLK
    mesh = plsc.VectorSubcoreMesh(core_axis_name="c", subcore_axis_name="s")

    def body(x_hbm, tab_hbm, out_hbm, idx_v, r0, r1, r2, r3,
             g0, g1, g2, g3, o0, o1, o2, o3):
        wid = lax.axis_index("s") * NC + lax.axis_index("c")
        base = wid * b_per_w
        pltpu.sync_copy(x_hbm.at[pl.ds(base, b_per_w)], idx_v)
        rows = [r0, r1, r2, r3]
        gsems = [g0, g1, g2, g3]
        osems = [o0, o1, o2, o3]

        def fire_gather(c, s):
            pltpu.async_copy(
                tab_hbm.at[idx_v.at[pl.ds(c * BLK, BLK)]], rows[s], gsems[s]
            )

        def drain_gather(c, s):
            pltpu.make_async_copy(
                tab_hbm.at[idx_v.at[pl.ds(c * BLK, BLK)]], rows[s], gsems[s]
            ).wait()

        def out_copy(c, s, fire):
            src = rows[s].at[pl.ds(0, BLK), pl.ds(0, D_OUT)]
            dst = out_hbm.at[pl.ds(base + c * BLK, BLK)]
            if fire:
                pltpu.async_copy(src, dst, osems[s])
            else:
                pltpu.make_async_copy(src, dst, osems[s]).wait()

        fire_gather(0, 0)
        fire_gather(1, 1)

        def group(g, carry):
            c0 = 4 * g
            for k in range(4):
                c = c0 + k

                @pl.when(c >= 2)
                def _():
                    out_copy(c - 2, (k + 2) % 4, False)

                @pl.when(c + 2 < n_blocks)
                def _():
                    fire_gather(c + 2, (k + 2) % 4)

                drain_gather(c, k)
                out_copy(c, k, True)
            return carry

        lax.fori_loop(0, n_blocks // 4, group, 0)
        out_copy(n_blocks - 2, (n_blocks - 2) % 4, False)
        out_copy(n_blocks - 1, (n_blocks - 1) % 4, False)

    return pl.kernel(
        body,
        out_type=jax.ShapeDtypeStruct((B, D_OUT), jnp.float32),
        mesh=mesh,
        compiler_params=pltpu.CompilerParams(
            use_tc_tiling_on_sc=False,
            needs_layout_passes=False,
            disable_bounds_checks=True,
        ),
        scratch_types=[
            pltpu.VMEM((b_per_w,), jnp.int32),
            pltpu.VMEM((BLK, 128), jnp.float32),
            pltpu.VMEM((BLK, 128), jnp.float32),
            pltpu.VMEM((BLK, 128), jnp.float32),
            pltpu.VMEM((BLK, 128), jnp.float32),
            pltpu.SemaphoreType.DMA,
            pltpu.SemaphoreType.DMA,
            pltpu.SemaphoreType.DMA,
            pltpu.SemaphoreType.DMA,
            pltpu.SemaphoreType.DMA,
            pltpu.SemaphoreType.DMA,
            pltpu.SemaphoreType.DMA,
            pltpu.SemaphoreType.DMA,
        ],
    )


@functools.cache
def _build_trans(S0: int, S1: int):
    NB = 16

    def body(in_ref, out_ref):
        eye = jnp.eye(D_OUT, dtype=jnp.float32)
        for b in range(NB):
            blk = in_ref[pl.ds(b * D_OUT, D_OUT), :]
            halves = [
                lax.dot_general(
                    blk[:, h * 64:(h + 1) * 64], eye,
                    (((0,), (0,)), ((), ())),
                    preferred_element_type=jnp.float32,
                )
                for h in range(2)
            ]
            y = jnp.concatenate(halves, axis=1)
            out_ref[0, :, b, :, :] = y.reshape(8, 8, BLK)

    return pl.pallas_call(
        body,
        grid=(S1, S0 // (BLK * NB)),
        in_specs=[
            pl.BlockSpec(
                (D_OUT * NB, 128),
                lambda s1, g: (s1 * (S0 // (BLK * NB)) + g, 0),
            )
        ],
        out_specs=pl.BlockSpec(
            (1, 8, NB, 8, BLK), lambda s1, g: (s1, 0, g, 0, 0)
        ),
        out_shape=jax.ShapeDtypeStruct(
            (S1, D_OUT // 8, S0 // BLK, 8, BLK), jnp.float32
        ),
    )


def kernel(x, table):
    s0, s1 = x.shape
    v, d = table.shape
    xt = x.T.reshape(s0 * s1).astype(jnp.int32)
    xt = xt.reshape(-1, 2, 64).transpose(0, 2, 1).reshape(s0 * s1)
    staged = _build_stage(v)(table.T)
    flat = _build_gather(s0 * s1, staged.shape[0])(xt, staged)
    out5d = _build_trans(s0, s1)(flat.reshape(s0 * s1 // 2, 128))
    return out5d.transpose(2, 4, 0, 1, 3).reshape(s0, s1, d)


# speedup vs baseline: 1.7321x; 1.4707x over previous
import functools

import jax
import jax.numpy as jnp
from jax import lax
from jax.experimental import pallas as pl
from jax.experimental.pallas import tpu as pltpu
from jax.experimental.pallas import tpu_sc as plsc

D_OUT = 64
SCALE = 8.0
LANES = 16
BLK = 128
TPAD = 129


@functools.cache
def _build(S0: int, S1: int, V: int):
    info = plsc.get_sparse_core_info()
    NC, NS = info.num_cores, info.num_subcores
    NW = NC * NS
    s0_blocks = S0 // BLK
    n_blocks = S1 * s0_blocks
    assert n_blocks % (2 * NW) == 0, (S0, S1, NW)
    blocks_per_w = n_blocks // NW
    n_pair = blocks_per_w // 2
    b_per_w = blocks_per_w * BLK
    mesh = plsc.VectorSubcoreMesh(core_axis_name="c", subcore_axis_name="s")

    def body(x_hbm, table_hbm, out_hbm, idx_v, rows_a, rows_b, rows_c, rows_d,
             trans_a, trans_b, sem_a, sem_b, sem_c, sem_d, sem_oa, sem_ob):
        wid = lax.axis_index("s") * NC + lax.axis_index("c")
        base = wid * b_per_w
        pltpu.sync_copy(x_hbm.at[pl.ds(base, b_per_w)], idx_v)
        iota16 = lax.iota(jnp.int32, 16)
        rowv = [iota16 + j0 for j0 in range(0, D_OUT, LANES)]

        def fire_gather(b, buf, sem):
            pltpu.async_copy(
                table_hbm.at[idx_v.at[pl.ds(b * BLK, BLK)]], buf, sem
            )

        def drain_gather(b, buf, sem):
            pltpu.make_async_copy(
                table_hbm.at[idx_v.at[pl.ds(b * BLK, BLK)]], buf, sem
            ).wait()

        def out_tiles(b, trans, sem, fire):
            blk_id = wid * blocks_per_w + b
            s1 = blk_id // s0_blocks
            s0b = lax.rem(blk_id, s0_blocks)
            for jb in range(D_OUT // 8):
                src = trans.at[pl.ds(jb * 8, 8), pl.ds(0, BLK)]
                dst = out_hbm.at[s1, jb, s0b]
                if fire:
                    pltpu.async_copy(src, dst, sem)
                else:
                    pltpu.make_async_copy(src, dst, sem).wait()

        def transpose_scale(rows, trans):
            def tok4(i, carry):
                t0 = i * 4
                for dt in range(4):
                    t = t0 + dt
                    colv = jnp.full((LANES,), 0, jnp.int32) + t
                    for k, j0 in enumerate(range(0, D_OUT, LANES)):
                        v = rows[t, pl.ds(j0, LANES)] * SCALE
                        plsc.store_scatter(trans, [rowv[k], colv], v)
                return carry

            lax.fori_loop(0, BLK // 4, tok4, 0)

        rows = [rows_a, rows_b, rows_c, rows_d]
        sems = [sem_a, sem_b, sem_c, sem_d]
        trans = [trans_a, trans_b]
        osems = [sem_oa, sem_ob]

        for b in range(4):
            fire_gather(b, rows[b], sems[b])

        def quad(g, carry):
            c0 = 4 * g
            for b in range(4):
                c = c0 + b
                drain_gather(c, rows[b], sems[b])
                p = b % 2

                @pl.when(c >= 2)
                def _():
                    out_tiles(c - 2, trans[p], osems[p], False)

                transpose_scale(rows[b], trans[p])
                out_tiles(c, trans[p], osems[p], True)

                @pl.when(c + 4 < blocks_per_w)
                def _():
                    fire_gather(c + 4, rows[b], sems[b])

            return carry

        lax.fori_loop(0, blocks_per_w // 4, quad, 0)
        out_tiles(blocks_per_w - 2, trans[0], osems[0], False)
        out_tiles(blocks_per_w - 1, trans[1], osems[1], False)

    return pl.kernel(
        body,
        out_type=jax.ShapeDtypeStruct(
            (S1, D_OUT // 8, S0 // BLK, 8, BLK), jnp.float32
        ),
        mesh=mesh,
        compiler_params=pltpu.CompilerParams(
            use_tc_tiling_on_sc=False,
            needs_layout_passes=False,
            disable_bounds_checks=True,
        ),
        scratch_types=[
            pltpu.VMEM((b_per_w,), jnp.int32),
            pltpu.VMEM((BLK, D_OUT), jnp.float32),
            pltpu.VMEM((BLK, D_OUT), jnp.float32),
            pltpu.VMEM((BLK, D_OUT), jnp.float32),
            pltpu.VMEM((BLK, D_OUT), jnp.float32),
            pltpu.VMEM((D_OUT, TPAD), jnp.float32),
            pltpu.VMEM((D_OUT, TPAD), jnp.float32),
            pltpu.SemaphoreType.DMA,
            pltpu.SemaphoreType.DMA,
            pltpu.SemaphoreType.DMA,
            pltpu.SemaphoreType.DMA,
            pltpu.SemaphoreType.DMA,
            pltpu.SemaphoreType.DMA,
        ],
    )


def kernel(x, table):
    s0, s1 = x.shape
    v, d = table.shape
    xt = x.T.reshape(s0 * s1).astype(jnp.int32)
    out5d = _build(s0, s1, v)(xt, table)
    return out5d.transpose(2, 4, 0, 1, 3).reshape(s0, s1, d)
